# speedup vs baseline: 2.9662x; 2.9662x over previous
import functools

import jax
import jax.numpy as jnp
from jax import lax
from jax.experimental import pallas as pl
from jax.experimental.pallas import tpu as pltpu
from jax.experimental.pallas import tpu_sc as plsc

V = 100000
E = 128
B = 4096
S = 50

NC = 2
NS = 16
NW = NC * NS
CHUNK = 128
ROWS = B * S
ROWS_PER_W = ROWS // NW
N_CHUNKS = ROWS_PER_W // CHUNK


def _gather_body(x_hbm, table_hbm, out_hbm, idx_v, rows_v, g_sem):
    wid = lax.axis_index("s") * NC + lax.axis_index("c")
    pltpu.sync_copy(x_hbm.at[wid], idx_v)
    base = wid * ROWS_PER_W

    def body(j, carry):
        pltpu.async_copy(table_hbm.at[idx_v.at[j]], rows_v, g_sem).wait()
        pltpu.sync_copy(rows_v, out_hbm.at[pl.ds(base + j * CHUNK, CHUNK)])
        return carry

    lax.fori_loop(0, N_CHUNKS, body, 0)


@jax.jit
def _run(x_flat, table):
    mesh = plsc.VectorSubcoreMesh(core_axis_name="c", subcore_axis_name="s")
    f = pl.kernel(
        _gather_body,
        mesh=mesh,
        out_type=jax.ShapeDtypeStruct((ROWS, E), jnp.float32),
        scratch_types=[
            pltpu.VMEM((N_CHUNKS, CHUNK), jnp.int32),
            pltpu.VMEM((CHUNK, E), jnp.float32),
            pltpu.SemaphoreType.DMA,
        ],
    )
    return f(x_flat, table)


def kernel(x, vocab):
    x_flat = x.reshape(NW, N_CHUNKS, CHUNK)
    table = vocab.reshape(V, E)
    out = _run(x_flat, table)
    return out.reshape(B, S, E)
